# baseline (speedup 1.0000x reference)
import functools
import jax
import jax.numpy as jnp
from jax.experimental import pallas as pl
from jax.experimental.pallas import tpu as pltpu

ENC = 512
NODES = 63
BATCH = 16
LEVELS = [
    [62, 46, 38, 54, 34, 42, 50, 58, 32, 36, 40, 44, 48, 52, 56, 60],
    [30, 22, 18, 26, 16, 20, 24, 28],
    [14, 10, 8, 12],
    [6, 4],
    [2],
    [0],
]
ALL_NODES = [nd for level in LEVELS for nd in level]

_DN_T = (((1,), (1,)), ((), ()))


def _body(tok_ref, emb_hbm, wih_hbm, whh_hbm, bih_ref, bhh_ref, sw_hbm,
          sb_ref, cw_ref, out_ref, x_vmem, wih_vmem, whh_vmem, sw_vmem,
          sem_rows, sem_wih, sem_whh, sem_sw):
    B = BATCH
    wih_copy = pltpu.make_async_copy(wih_hbm, wih_vmem, sem_wih)
    wih_copy.start()

    chunk_of_level = [0, 2, 3, 3, 3, 3]
    chunk_copies = [[], [], [], []]
    r = 0
    for s, level in enumerate(LEVELS):
        for pos, node in enumerate(level):
            chunk = (0 if pos < 8 else 1) if s == 0 else chunk_of_level[s]
            for b in range(B):
                tok = tok_ref[b, node]
                cp = pltpu.make_async_copy(emb_hbm.at[pl.ds(tok, 1)],
                                           x_vmem.at[pl.ds(r, 1)],
                                           sem_rows.at[chunk])
                cp.start()
                chunk_copies[chunk].append(cp)
                r += 1

    whh_copy = pltpu.make_async_copy(whh_hbm, whh_vmem, sem_whh)
    sw_copy = pltpu.make_async_copy(sw_hbm, sw_vmem, sem_sw)
    whh_copy.start()
    sw_copy.start()
    wih_copy.wait()

    b_ih = jnp.reshape(bih_ref[...], (1, 3 * ENC))
    b_hh = jnp.reshape(bhh_ref[...], (1, 3 * ENC))
    sb = sb_ref[...]
    cw = cw_ref[...]
    c0 = jnp.dot(jnp.tanh(sb), cw, preferred_element_type=jnp.float32)

    CH = 8 * B

    def gi_chunk(ci):
        for cp in chunk_copies[ci]:
            cp.wait()
        return jax.lax.dot_general(x_vmem[ci * CH:(ci + 1) * CH],
                                   wih_vmem[...], _DN_T,
                                   preferred_element_type=jnp.float32) + b_ih

    def gru(gi, gh, c):
        i_r = gi[:, 0:ENC]
        i_z = gi[:, ENC:2 * ENC]
        i_n = gi[:, 2 * ENC:3 * ENC]
        h_r = gh[:, 0:ENC]
        h_z = gh[:, ENC:2 * ENC]
        h_n = gh[:, 2 * ENC:3 * ENC]
        rr = jax.nn.sigmoid(i_r + h_r)
        z = jax.nn.sigmoid(i_z + h_z)
        nn_ = jnp.tanh(i_n + rr * h_n)
        return (1.0 - z) * nn_ + z * c

    def maxacc(out, h):
        for i in range(h.shape[0] // B):
            out = jnp.maximum(out, h[i * B:(i + 1) * B])
        return out

    out = jnp.zeros((B, ENC), dtype=jnp.float32)

    gh_leaf = jnp.broadcast_to(b_hh, (CH, 3 * ENC))
    c_leaf = jnp.zeros((CH, ENC), dtype=jnp.float32)
    h0 = gru(gi_chunk(0), gh_leaf, c_leaf)
    out = maxacc(out, h0)
    h1 = gru(gi_chunk(1), gh_leaf, c_leaf)
    out = maxacc(out, h1)

    sw_copy.wait()
    whh_copy.wait()
    sw = sw_vmem[...]

    def internal_level(gi, h_child, is_root):
        t = jnp.tanh(jnp.dot(h_child, sw,
                             preferred_element_type=jnp.float32) + sb)
        hW = jax.lax.dot_general(h_child, whh_vmem[...], _DN_T,
                                 preferred_element_type=jnp.float32)
        l = jnp.tanh(jnp.dot(t, cw, preferred_element_type=jnp.float32))
        k = 15.0 if is_root else 1.0
        gate = 1.0 / (1.0 + k * jnp.exp(c0 - l))
        c = h_child * gate
        gh = gate * hW + b_hh
        return gru(gi, gh, c)

    h2 = internal_level(gi_chunk(2), h0, False)
    out = maxacc(out, h2)

    gi36 = gi_chunk(3)
    h_prev = h2
    off = 0
    for s in range(2, len(LEVELS)):
        n = len(LEVELS[s]) * B
        h = internal_level(gi36[off:off + n], h_prev[:n],
                           s == len(LEVELS) - 1)
        out = maxacc(out, h)
        h_prev = h
        off += n
    out_ref[...] = jnp.maximum(out, 0.0)


@jax.jit
def _run(tokens, emb, W_ih, W_hh, b_ih, b_hh, sent_weight, sent_bias,
         context_weight):
    vm = pltpu.MemorySpace.VMEM
    hbm = pltpu.MemorySpace.HBM
    smem = pltpu.MemorySpace.SMEM
    out = pl.pallas_call(
        _body,
        in_specs=[
            pl.BlockSpec(memory_space=smem),
            pl.BlockSpec(memory_space=hbm),
            pl.BlockSpec(memory_space=hbm),
            pl.BlockSpec(memory_space=hbm),
            pl.BlockSpec(memory_space=vm),
            pl.BlockSpec(memory_space=vm),
            pl.BlockSpec(memory_space=hbm),
            pl.BlockSpec(memory_space=vm),
            pl.BlockSpec(memory_space=vm),
        ],
        scratch_shapes=[
            pltpu.VMEM((32 * BATCH, ENC), jnp.float32),
            pltpu.VMEM((3 * ENC, ENC), jnp.float32),
            pltpu.VMEM((3 * ENC, ENC), jnp.float32),
            pltpu.VMEM((ENC, ENC), jnp.float32),
            pltpu.SemaphoreType.DMA((4,)),
            pltpu.SemaphoreType.DMA,
            pltpu.SemaphoreType.DMA,
            pltpu.SemaphoreType.DMA,
        ],
        out_shape=jax.ShapeDtypeStruct((BATCH, ENC), jnp.float32),
    )(tokens, emb, W_ih, W_hh, b_ih, b_hh, sent_weight, sent_bias,
      context_weight)
    return out


def kernel(tokens, bs, emb, W_ih, W_hh, b_ih, b_hh, sent_weight, sent_bias,
           context_weight):
    del bs
    return _run(tokens, emb, W_ih, W_hh, b_ih, b_hh, sent_weight,
                sent_bias, context_weight)


# speedup vs baseline: 1.0997x; 1.0997x over previous
import functools
import jax
import jax.numpy as jnp
from jax.experimental import pallas as pl
from jax.experimental.pallas import tpu as pltpu

ENC = 512
NODES = 63
BATCH = 16
LEVELS = [
    [62, 46, 38, 54, 34, 42, 50, 58, 32, 36, 40, 44, 48, 52, 56, 60],
    [30, 22, 18, 26, 16, 20, 24, 28],
    [14, 10, 8, 12],
    [6, 4],
    [2],
    [0],
]
ALL_NODES = [nd for level in LEVELS for nd in level]

_DN_T = (((1,), (1,)), ((), ()))


def _body(tok_ref, emb_hbm, wih_hbm, whh_hbm, bih_ref, bhh_ref, sw_hbm,
          sb_ref, cw_ref, out_ref, x_vmem, wih_vmem, whh_vmem, sw_vmem,
          sem_rows, sem_wih, sem_whh, sem_sw):
    B = BATCH
    wih_copy = pltpu.make_async_copy(wih_hbm, wih_vmem, sem_wih)
    wih_copy.start()

    chunk_copies = [[], []]
    r = 0
    for s, level in enumerate(LEVELS):
        chunk = 0 if s == 0 else 1
        for node in level:
            for b in range(B):
                tok = tok_ref[b, node]
                cp = pltpu.make_async_copy(emb_hbm.at[pl.ds(tok, 1)],
                                           x_vmem.at[pl.ds(r, 1)],
                                           sem_rows.at[chunk])
                cp.start()
                chunk_copies[chunk].append(cp)
                r += 1

    whh_copy = pltpu.make_async_copy(whh_hbm, whh_vmem, sem_whh)
    sw_copy = pltpu.make_async_copy(sw_hbm, sw_vmem, sem_sw)
    whh_copy.start()
    sw_copy.start()
    wih_copy.wait()

    b_ih = jnp.reshape(bih_ref[...], (1, 3 * ENC))
    b_hh = jnp.reshape(bhh_ref[...], (1, 3 * ENC))
    sb = sb_ref[...]
    cw = cw_ref[...]
    c0 = jnp.dot(jnp.tanh(sb), cw, preferred_element_type=jnp.float32)

    CH = 16 * B

    def gi_chunk(ci):
        for cp in chunk_copies[ci]:
            cp.wait()
        return jax.lax.dot_general(x_vmem[ci * CH:(ci + 1) * CH],
                                   wih_vmem[...], _DN_T,
                                   preferred_element_type=jnp.float32) + b_ih

    def gru(gi, gh, c):
        i_r = gi[:, 0:ENC]
        i_z = gi[:, ENC:2 * ENC]
        i_n = gi[:, 2 * ENC:3 * ENC]
        h_r = gh[:, 0:ENC]
        h_z = gh[:, ENC:2 * ENC]
        h_n = gh[:, 2 * ENC:3 * ENC]
        rr = jax.nn.sigmoid(i_r + h_r)
        z = jax.nn.sigmoid(i_z + h_z)
        nn_ = jnp.tanh(i_n + rr * h_n)
        return (1.0 - z) * nn_ + z * c

    def maxacc(out, h):
        for i in range(h.shape[0] // B):
            out = jnp.maximum(out, h[i * B:(i + 1) * B])
        return out

    out = jnp.zeros((B, ENC), dtype=jnp.float32)

    gh_leaf = jnp.broadcast_to(b_hh, (CH, 3 * ENC))
    c_leaf = jnp.zeros((CH, ENC), dtype=jnp.float32)
    h_leaf = gru(gi_chunk(0), gh_leaf, c_leaf)
    out = maxacc(out, h_leaf)

    sw_copy.wait()
    whh_copy.wait()
    sw = sw_vmem[...]

    def internal_level(gi, h_child, is_root):
        t = jnp.tanh(jnp.dot(h_child, sw,
                             preferred_element_type=jnp.float32) + sb)
        hW = jax.lax.dot_general(h_child, whh_vmem[...], _DN_T,
                                 preferred_element_type=jnp.float32)
        l = jnp.tanh(jnp.dot(t, cw, preferred_element_type=jnp.float32))
        k = 15.0 if is_root else 1.0
        gate = 1.0 / (1.0 + k * jnp.exp(c0 - l))
        c = h_child * gate
        gh = gate * hW + b_hh
        return gru(gi, gh, c)

    gi_b = gi_chunk(1)
    h_prev = h_leaf
    off = 0
    for s in range(1, len(LEVELS)):
        n = len(LEVELS[s]) * B
        h = internal_level(gi_b[off:off + n], h_prev[:n],
                           s == len(LEVELS) - 1)
        out = maxacc(out, h)
        h_prev = h
        off += n
    out_ref[...] = jnp.maximum(out, 0.0)


@jax.jit
def _run(tokens, emb, W_ih, W_hh, b_ih, b_hh, sent_weight, sent_bias,
         context_weight):
    vm = pltpu.MemorySpace.VMEM
    hbm = pltpu.MemorySpace.HBM
    smem = pltpu.MemorySpace.SMEM
    out = pl.pallas_call(
        _body,
        in_specs=[
            pl.BlockSpec(memory_space=smem),
            pl.BlockSpec(memory_space=hbm),
            pl.BlockSpec(memory_space=hbm),
            pl.BlockSpec(memory_space=hbm),
            pl.BlockSpec(memory_space=vm),
            pl.BlockSpec(memory_space=vm),
            pl.BlockSpec(memory_space=hbm),
            pl.BlockSpec(memory_space=vm),
            pl.BlockSpec(memory_space=vm),
        ],
        scratch_shapes=[
            pltpu.VMEM((32 * BATCH, ENC), jnp.float32),
            pltpu.VMEM((3 * ENC, ENC), jnp.float32),
            pltpu.VMEM((3 * ENC, ENC), jnp.float32),
            pltpu.VMEM((ENC, ENC), jnp.float32),
            pltpu.SemaphoreType.DMA((2,)),
            pltpu.SemaphoreType.DMA,
            pltpu.SemaphoreType.DMA,
            pltpu.SemaphoreType.DMA,
        ],
        out_shape=jax.ShapeDtypeStruct((BATCH, ENC), jnp.float32),
    )(tokens, emb, W_ih, W_hh, b_ih, b_hh, sent_weight, sent_bias,
      context_weight)
    return out


def kernel(tokens, bs, emb, W_ih, W_hh, b_ih, b_hh, sent_weight, sent_bias,
           context_weight):
    del bs
    return _run(tokens, emb, W_ih, W_hh, b_ih, b_hh, sent_weight,
                sent_bias, context_weight)


# speedup vs baseline: 1.1266x; 1.0245x over previous
import functools
import jax
import jax.numpy as jnp
from jax.experimental import pallas as pl
from jax.experimental.pallas import tpu as pltpu

ENC = 512
NODES = 63
BATCH = 16
LEVELS = [
    [62, 46, 38, 54, 34, 42, 50, 58, 32, 36, 40, 44, 48, 52, 56, 60],
    [30, 22, 18, 26, 16, 20, 24, 28],
    [14, 10, 8, 12],
    [6, 4],
    [2],
    [0],
]
ALL_NODES = [nd for level in LEVELS for nd in level]

_DN_T = (((1,), (1,)), ((), ()))


def _body(tok_vmem, emb_hbm, wih_hbm, whh_hbm, bih_ref, bhh_ref, sw_hbm,
          sb_ref, cw_ref, out_ref, x_vmem, wih_vmem, whh_vmem, sw_vmem,
          tok_smem, sem_rows, sem_wih, sem_whh, sem_sw, sem_tok):
    B = BATCH
    wih_copy = pltpu.make_async_copy(wih_hbm, wih_vmem, sem_wih)
    wih_copy.start()

    tok_copy = pltpu.make_async_copy(tok_vmem, tok_smem, sem_tok)
    tok_copy.start()
    tok_copy.wait()
    tok_ref = tok_smem

    chunk_copies = [[], []]
    r = 0
    for s, level in enumerate(LEVELS):
        chunk = 0 if s == 0 else 1
        for node in level:
            for b in range(B):
                tok = tok_ref[b, node]
                cp = pltpu.make_async_copy(emb_hbm.at[pl.ds(tok, 1)],
                                           x_vmem.at[pl.ds(r, 1)],
                                           sem_rows.at[chunk])
                cp.start()
                chunk_copies[chunk].append(cp)
                r += 1

    whh_copy = pltpu.make_async_copy(whh_hbm, whh_vmem, sem_whh)
    sw_copy = pltpu.make_async_copy(sw_hbm, sw_vmem, sem_sw)
    whh_copy.start()
    sw_copy.start()
    wih_copy.wait()

    b_ih = jnp.reshape(bih_ref[...], (1, 3 * ENC))
    b_hh = jnp.reshape(bhh_ref[...], (1, 3 * ENC))
    sb = sb_ref[...]
    cw = cw_ref[...]
    c0 = jnp.dot(jnp.tanh(sb), cw, preferred_element_type=jnp.float32)

    CH = 16 * B

    def gi_chunk(ci):
        for cp in chunk_copies[ci]:
            cp.wait()
        return jax.lax.dot_general(x_vmem[ci * CH:(ci + 1) * CH],
                                   wih_vmem[...], _DN_T,
                                   preferred_element_type=jnp.float32) + b_ih

    def gru(gi, gh, c):
        i_r = gi[:, 0:ENC]
        i_z = gi[:, ENC:2 * ENC]
        i_n = gi[:, 2 * ENC:3 * ENC]
        h_r = gh[:, 0:ENC]
        h_z = gh[:, ENC:2 * ENC]
        h_n = gh[:, 2 * ENC:3 * ENC]
        rr = jax.nn.sigmoid(i_r + h_r)
        z = jax.nn.sigmoid(i_z + h_z)
        nn_ = jnp.tanh(i_n + rr * h_n)
        return (1.0 - z) * nn_ + z * c

    def maxacc(out, h):
        for i in range(h.shape[0] // B):
            out = jnp.maximum(out, h[i * B:(i + 1) * B])
        return out

    out = jnp.zeros((B, ENC), dtype=jnp.float32)

    gh_leaf = jnp.broadcast_to(b_hh, (CH, 3 * ENC))
    c_leaf = jnp.zeros((CH, ENC), dtype=jnp.float32)
    h_leaf = gru(gi_chunk(0), gh_leaf, c_leaf)
    out = maxacc(out, h_leaf)

    sw_copy.wait()
    whh_copy.wait()
    sw = sw_vmem[...]

    def internal_level(gi, h_child, is_root):
        t = jnp.tanh(jnp.dot(h_child, sw,
                             preferred_element_type=jnp.float32) + sb)
        hW = jax.lax.dot_general(h_child, whh_vmem[...], _DN_T,
                                 preferred_element_type=jnp.float32)
        l = jnp.tanh(jnp.dot(t, cw, preferred_element_type=jnp.float32))
        k = 15.0 if is_root else 1.0
        gate = 1.0 / (1.0 + k * jnp.exp(c0 - l))
        c = h_child * gate
        gh = gate * hW + b_hh
        return gru(gi, gh, c)

    gi_b = gi_chunk(1)
    h_prev = h_leaf
    off = 0
    for s in range(1, len(LEVELS)):
        n = len(LEVELS[s]) * B
        h = internal_level(gi_b[off:off + n], h_prev[:n],
                           s == len(LEVELS) - 1)
        out = maxacc(out, h)
        h_prev = h
        off += n
    out_ref[...] = jnp.maximum(out, 0.0)


@jax.jit
def _run(tokens, emb, W_ih, W_hh, b_ih, b_hh, sent_weight, sent_bias,
         context_weight):
    vm = pltpu.MemorySpace.VMEM
    hbm = pltpu.MemorySpace.HBM
    smem = pltpu.MemorySpace.SMEM
    out = pl.pallas_call(
        _body,
        in_specs=[
            pl.BlockSpec(memory_space=vm),
            pl.BlockSpec(memory_space=hbm),
            pl.BlockSpec(memory_space=hbm),
            pl.BlockSpec(memory_space=hbm),
            pl.BlockSpec(memory_space=vm),
            pl.BlockSpec(memory_space=vm),
            pl.BlockSpec(memory_space=hbm),
            pl.BlockSpec(memory_space=vm),
            pl.BlockSpec(memory_space=vm),
        ],
        scratch_shapes=[
            pltpu.VMEM((32 * BATCH, ENC), jnp.float32),
            pltpu.VMEM((3 * ENC, ENC), jnp.float32),
            pltpu.VMEM((3 * ENC, ENC), jnp.float32),
            pltpu.VMEM((ENC, ENC), jnp.float32),
            pltpu.SMEM((BATCH, NODES), jnp.int32),
            pltpu.SemaphoreType.DMA((2,)),
            pltpu.SemaphoreType.DMA,
            pltpu.SemaphoreType.DMA,
            pltpu.SemaphoreType.DMA,
            pltpu.SemaphoreType.DMA,
        ],
        out_shape=jax.ShapeDtypeStruct((BATCH, ENC), jnp.float32),
    )(tokens, emb, W_ih, W_hh, b_ih, b_hh, sent_weight, sent_bias,
      context_weight)
    return out


def kernel(tokens, bs, emb, W_ih, W_hh, b_ih, b_hh, sent_weight, sent_bias,
           context_weight):
    del bs
    return _run(tokens, emb, W_ih, W_hh, b_ih, b_hh, sent_weight,
                sent_bias, context_weight)
